# initial kernel scaffold (unmeasured)
import functools

import jax
import jax.numpy as jnp
from jax import lax
from jax.experimental import pallas as pl
from jax.experimental.pallas import tpu as pltpu

N_DEV = 8
M = 3072
M_CHUNK = M // N_DEV


def kernel(A, B):
    m, k = A.shape
    k2, n = B.shape
    assert m == M and k == k2

    def body(a_ref, b_ref, out_ref, acc_ref, pp_ref, send_sems, recv_sems,
             credit_sem):
        my = lax.axis_index("i")
        left = lax.rem(my + (N_DEV - 1), N_DEV)
        right = lax.rem(my + 1, N_DEV)

        barrier_sem = pltpu.get_barrier_semaphore()
        for nbr in (left, right):
            pl.semaphore_signal(
                barrier_sem, inc=1,
                device_id=(nbr,), device_id_type=pl.DeviceIdType.MESH,
            )
        pl.semaphore_wait(barrier_sem, 2)

        def chunk_rows(c):
            return pl.ds(c * M_CHUNK, M_CHUNK)

        def local_partial(c, dst):
            dst[...] = jnp.dot(
                a_ref[chunk_rows(c), :], b_ref[...],
                preferred_element_type=jnp.float32,
            )

        c0 = lax.rem(my + (N_DEV - 1), N_DEV)
        local_partial(c0, acc_ref.at[0])

        for s in range(N_DEV - 1):
            send_slot = s % 2
            recv_slot = (s + 1) % 2
            if s >= 2:
                pl.semaphore_wait(credit_sem, 1)
            rdma = pltpu.make_async_remote_copy(
                src_ref=acc_ref.at[send_slot],
                dst_ref=acc_ref.at[recv_slot],
                send_sem=send_sems.at[send_slot],
                recv_sem=recv_sems.at[recv_slot],
                device_id=(right,),
                device_id_type=pl.DeviceIdType.MESH,
            )
            rdma.start()
            c = lax.rem(my + (2 * N_DEV - 2 - s), N_DEV)
            local_partial(c, pp_ref)
            rdma.wait()
            pl.semaphore_signal(
                credit_sem, inc=1,
                device_id=(left,), device_id_type=pl.DeviceIdType.MESH,
            )
            if s < N_DEV - 2:
                acc_ref[recv_slot] = acc_ref[recv_slot] + pp_ref[...]
            else:
                out_ref[...] = acc_ref[recv_slot] + pp_ref[...]

        pl.semaphore_wait(credit_sem, 2)

    return pl.pallas_call(
        body,
        out_shape=jax.ShapeDtypeStruct((M_CHUNK, n), jnp.float32),
        in_specs=[
            pl.BlockSpec(memory_space=pltpu.VMEM),
            pl.BlockSpec(memory_space=pltpu.VMEM),
        ],
        out_specs=pl.BlockSpec(memory_space=pltpu.VMEM),
        scratch_shapes=[
            pltpu.VMEM((2, M_CHUNK, n), jnp.float32),
            pltpu.VMEM((M_CHUNK, n), jnp.float32),
            pltpu.SemaphoreType.DMA((2,)),
            pltpu.SemaphoreType.DMA((2,)),
            pltpu.SemaphoreType.REGULAR,
        ],
        compiler_params=pltpu.CompilerParams(collective_id=0),
    )(A, B)


# baseline (device time: 396581 ns/iter reference)
import functools

import jax
import jax.numpy as jnp
from jax import lax
from jax.experimental import pallas as pl
from jax.experimental.pallas import tpu as pltpu

N_DEV = 8
M = 3072
M_CHUNK = M // N_DEV


def kernel(A, B):
    m, k = A.shape
    k2, n = B.shape
    assert m == M and k == k2

    def body(a_ref, b_ref, out_ref, acc_ref, pp_ref, send_sems, recv_sems,
             credit_sem):
        my = lax.axis_index("i")
        left = lax.rem(my + (N_DEV - 1), N_DEV)
        right = lax.rem(my + 1, N_DEV)

        barrier_sem = pltpu.get_barrier_semaphore()
        for nbr in (left, right):
            pl.semaphore_signal(
                barrier_sem, inc=1,
                device_id=(nbr,), device_id_type=pl.DeviceIdType.MESH,
            )
        pl.semaphore_wait(barrier_sem, 2)

        def chunk_rows(c):
            return pl.ds(c * M_CHUNK, M_CHUNK)

        def local_partial(c, dst):
            dst[...] = jnp.dot(
                a_ref[chunk_rows(c), :], b_ref[...],
                preferred_element_type=jnp.float32,
            )

        c0 = lax.rem(my + (N_DEV - 1), N_DEV)
        local_partial(c0, acc_ref.at[0])

        for s in range(N_DEV - 1):
            send_slot = s % 2
            recv_slot = (s + 1) % 2
            if s >= 2:
                pl.semaphore_wait(credit_sem, 1)
            rdma = pltpu.make_async_remote_copy(
                src_ref=acc_ref.at[send_slot],
                dst_ref=acc_ref.at[recv_slot],
                send_sem=send_sems.at[send_slot],
                recv_sem=recv_sems.at[recv_slot],
                device_id=(right,),
                device_id_type=pl.DeviceIdType.MESH,
            )
            rdma.start()
            c = lax.rem(my + (2 * N_DEV - 2 - s), N_DEV)
            local_partial(c, pp_ref)
            rdma.wait()
            pl.semaphore_signal(
                credit_sem, inc=1,
                device_id=(left,), device_id_type=pl.DeviceIdType.MESH,
            )
            if s < N_DEV - 2:
                acc_ref[recv_slot] = acc_ref[recv_slot] + pp_ref[...]
            else:
                out_ref[...] = acc_ref[recv_slot] + pp_ref[...]

        pl.semaphore_wait(credit_sem, 2)

    return pl.pallas_call(
        body,
        out_shape=jax.ShapeDtypeStruct((M_CHUNK, n), jnp.float32),
        in_specs=[
            pl.BlockSpec(memory_space=pltpu.VMEM),
            pl.BlockSpec(memory_space=pltpu.VMEM),
        ],
        out_specs=pl.BlockSpec(memory_space=pltpu.VMEM),
        scratch_shapes=[
            pltpu.VMEM((2, M_CHUNK, n), jnp.float32),
            pltpu.VMEM((M_CHUNK, n), jnp.float32),
            pltpu.SemaphoreType.DMA((2,)),
            pltpu.SemaphoreType.DMA((2,)),
            pltpu.SemaphoreType.REGULAR,
        ],
        compiler_params=pltpu.CompilerParams(
            collective_id=0,
            vmem_limit_bytes=100 * 1024 * 1024,
        ),
    )(A, B)


# device time: 223312 ns/iter; 1.7759x vs baseline; 1.7759x over previous
import jax
import jax.numpy as jnp
from jax import lax
from jax.experimental import pallas as pl
from jax.experimental.pallas import tpu as pltpu

N_DEV = 8
M = 3072
M_CHUNK = M // N_DEV


def kernel(A, B):
    m, k = A.shape
    k2, n = B.shape
    assert m == M and k == k2
    nh = n // 2

    def body(a_ref, b_ref, out_ref, accf_ref, accr_ref, ppf_ref, ppr_ref,
             send_f, recv_f, send_r, recv_r, credit_f, credit_r):
        my = lax.axis_index("i")
        left = lax.rem(my + (N_DEV - 1), N_DEV)
        right = lax.rem(my + 1, N_DEV)

        barrier_sem = pltpu.get_barrier_semaphore()
        for nbr in (left, right):
            pl.semaphore_signal(
                barrier_sem, inc=1,
                device_id=(nbr,), device_id_type=pl.DeviceIdType.MESH,
            )
        pl.semaphore_wait(barrier_sem, 2)

        def rows(c):
            return pl.ds(c * M_CHUNK, M_CHUNK)

        def partial_f(c, dst):
            dst[...] = jnp.dot(
                a_ref[rows(c), :], b_ref[:, :nh],
                preferred_element_type=jnp.float32,
            )

        def partial_r(c, dst):
            dst[...] = jnp.dot(
                a_ref[rows(c), :], b_ref[:, nh:],
                preferred_element_type=jnp.float32,
            )

        partial_f(lax.rem(my + (N_DEV - 1), N_DEV), accf_ref.at[0])
        partial_r(lax.rem(my + 1, N_DEV), accr_ref.at[0])

        for s in range(N_DEV - 1):
            snd = s % 2
            rcv = (s + 1) % 2
            if s >= 2:
                pl.semaphore_wait(credit_f, 1)
                pl.semaphore_wait(credit_r, 1)
            rdma_f = pltpu.make_async_remote_copy(
                src_ref=accf_ref.at[snd], dst_ref=accf_ref.at[rcv],
                send_sem=send_f.at[snd], recv_sem=recv_f.at[rcv],
                device_id=(right,), device_id_type=pl.DeviceIdType.MESH,
            )
            rdma_r = pltpu.make_async_remote_copy(
                src_ref=accr_ref.at[snd], dst_ref=accr_ref.at[rcv],
                send_sem=send_r.at[snd], recv_sem=recv_r.at[rcv],
                device_id=(left,), device_id_type=pl.DeviceIdType.MESH,
            )
            rdma_f.start()
            rdma_r.start()
            cf = lax.rem(my + (2 * N_DEV - 2 - s), N_DEV)
            cr = lax.rem(my + 2 + s, N_DEV)
            partial_f(cf, ppf_ref)
            partial_r(cr, ppr_ref)
            rdma_f.wait()
            rdma_r.wait()
            pl.semaphore_signal(
                credit_f, inc=1,
                device_id=(left,), device_id_type=pl.DeviceIdType.MESH,
            )
            pl.semaphore_signal(
                credit_r, inc=1,
                device_id=(right,), device_id_type=pl.DeviceIdType.MESH,
            )
            if s < N_DEV - 2:
                accf_ref[rcv] = accf_ref[rcv] + ppf_ref[...]
                accr_ref[rcv] = accr_ref[rcv] + ppr_ref[...]
            else:
                out_ref[:, :nh] = accf_ref[rcv] + ppf_ref[...]
                out_ref[:, nh:] = accr_ref[rcv] + ppr_ref[...]

        pl.semaphore_wait(credit_f, 2)
        pl.semaphore_wait(credit_r, 2)

    return pl.pallas_call(
        body,
        out_shape=jax.ShapeDtypeStruct((M_CHUNK, n), jnp.float32),
        in_specs=[
            pl.BlockSpec(memory_space=pltpu.VMEM),
            pl.BlockSpec(memory_space=pltpu.VMEM),
        ],
        out_specs=pl.BlockSpec(memory_space=pltpu.VMEM),
        scratch_shapes=[
            pltpu.VMEM((2, M_CHUNK, nh), jnp.float32),
            pltpu.VMEM((2, M_CHUNK, nh), jnp.float32),
            pltpu.VMEM((M_CHUNK, nh), jnp.float32),
            pltpu.VMEM((M_CHUNK, nh), jnp.float32),
            pltpu.SemaphoreType.DMA((2,)),
            pltpu.SemaphoreType.DMA((2,)),
            pltpu.SemaphoreType.DMA((2,)),
            pltpu.SemaphoreType.DMA((2,)),
            pltpu.SemaphoreType.REGULAR,
            pltpu.SemaphoreType.REGULAR,
        ],
        compiler_params=pltpu.CompilerParams(
            collective_id=0,
            vmem_limit_bytes=100 * 1024 * 1024,
        ),
    )(A, B)


# device time: 149161 ns/iter; 2.6587x vs baseline; 1.4971x over previous
import jax
import jax.numpy as jnp
from jax import lax
from jax.experimental import pallas as pl
from jax.experimental.pallas import tpu as pltpu

N_DEV = 8
M = 3072
M_CHUNK = M // N_DEV


def kernel(A, B):
    m, k = A.shape
    k2, n = B.shape
    assert m == M and k == k2
    nh = n // 2

    def body(a16_ref, b16_ref, out_ref, accf_ref, accr_ref,
             ppf_ref, ppr_ref, send_f, recv_f, send_r, recv_r,
             credit_f, credit_r):
        my = lax.axis_index("i")
        left = lax.rem(my + (N_DEV - 1), N_DEV)
        right = lax.rem(my + 1, N_DEV)

        barrier_sem = pltpu.get_barrier_semaphore()
        for nbr in (left, right):
            pl.semaphore_signal(
                barrier_sem, inc=1,
                device_id=(nbr,), device_id_type=pl.DeviceIdType.MESH,
            )
        pl.semaphore_wait(barrier_sem, 2)

        def rows(c):
            return pl.ds(c * M_CHUNK, M_CHUNK)

        def partial_f(c, dst):
            dst[...] = jnp.dot(
                a16_ref[rows(c), :], b16_ref[:, :nh],
                preferred_element_type=jnp.float32,
            )

        def partial_r(c, dst):
            dst[...] = jnp.dot(
                a16_ref[rows(c), :], b16_ref[:, nh:],
                preferred_element_type=jnp.float32,
            )

        partial_f(lax.rem(my + (N_DEV - 1), N_DEV), ppf_ref)
        partial_r(lax.rem(my + 1, N_DEV), ppr_ref)
        accf_ref[0] = ppf_ref[...].astype(jnp.bfloat16)
        accr_ref[0] = ppr_ref[...].astype(jnp.bfloat16)

        for s in range(N_DEV - 1):
            snd = s % 2
            rcv = (s + 1) % 2
            if s >= 2:
                pl.semaphore_wait(credit_f, 1)
                pl.semaphore_wait(credit_r, 1)
            rdma_f = pltpu.make_async_remote_copy(
                src_ref=accf_ref.at[snd], dst_ref=accf_ref.at[rcv],
                send_sem=send_f.at[snd], recv_sem=recv_f.at[rcv],
                device_id=(right,), device_id_type=pl.DeviceIdType.MESH,
            )
            rdma_r = pltpu.make_async_remote_copy(
                src_ref=accr_ref.at[snd], dst_ref=accr_ref.at[rcv],
                send_sem=send_r.at[snd], recv_sem=recv_r.at[rcv],
                device_id=(left,), device_id_type=pl.DeviceIdType.MESH,
            )
            rdma_f.start()
            rdma_r.start()
            cf = lax.rem(my + (2 * N_DEV - 2 - s), N_DEV)
            cr = lax.rem(my + 2 + s, N_DEV)
            partial_f(cf, ppf_ref)
            partial_r(cr, ppr_ref)
            rdma_f.wait()
            rdma_r.wait()
            pl.semaphore_signal(
                credit_f, inc=1,
                device_id=(left,), device_id_type=pl.DeviceIdType.MESH,
            )
            pl.semaphore_signal(
                credit_r, inc=1,
                device_id=(right,), device_id_type=pl.DeviceIdType.MESH,
            )
            if s < N_DEV - 2:
                accf_ref[rcv] = (
                    accf_ref[rcv].astype(jnp.float32) + ppf_ref[...]
                ).astype(jnp.bfloat16)
                accr_ref[rcv] = (
                    accr_ref[rcv].astype(jnp.float32) + ppr_ref[...]
                ).astype(jnp.bfloat16)
            else:
                out_ref[:, :nh] = accf_ref[rcv].astype(jnp.float32) + ppf_ref[...]
                out_ref[:, nh:] = accr_ref[rcv].astype(jnp.float32) + ppr_ref[...]

        pl.semaphore_wait(credit_f, 2)
        pl.semaphore_wait(credit_r, 2)

    return pl.pallas_call(
        body,
        out_shape=jax.ShapeDtypeStruct((M_CHUNK, n), jnp.float32),
        in_specs=[
            pl.BlockSpec(memory_space=pltpu.VMEM),
            pl.BlockSpec(memory_space=pltpu.VMEM),
        ],
        out_specs=pl.BlockSpec(memory_space=pltpu.VMEM),
        scratch_shapes=[
            pltpu.VMEM((2, M_CHUNK, nh), jnp.bfloat16),
            pltpu.VMEM((2, M_CHUNK, nh), jnp.bfloat16),
            pltpu.VMEM((M_CHUNK, nh), jnp.float32),
            pltpu.VMEM((M_CHUNK, nh), jnp.float32),
            pltpu.SemaphoreType.DMA((2,)),
            pltpu.SemaphoreType.DMA((2,)),
            pltpu.SemaphoreType.DMA((2,)),
            pltpu.SemaphoreType.DMA((2,)),
            pltpu.SemaphoreType.REGULAR,
            pltpu.SemaphoreType.REGULAR,
        ],
        compiler_params=pltpu.CompilerParams(
            collective_id=0,
            vmem_limit_bytes=100 * 1024 * 1024,
        ),
    )(A.astype(jnp.bfloat16), B.astype(jnp.bfloat16))


# device time: 138314 ns/iter; 2.8673x vs baseline; 1.0784x over previous
import jax
import jax.numpy as jnp
from jax import lax
from jax.experimental import pallas as pl
from jax.experimental.pallas import tpu as pltpu

N_DEV = 8
M = 3072
M_CHUNK = M // N_DEV

N_CVT = 8


def kernel(A, B):
    m, k = A.shape
    k2, n = B.shape
    assert m == M and k == k2
    nh = n // 2
    a_rows = m // N_CVT
    b_rows = k // N_CVT

    def body(a_hbm, b_hbm, out_ref, a16_ref, b16_ref, sta_ref, stb_ref,
             accf_ref, accr_ref, ppf_ref, ppr_ref,
             cp_sems_a, cp_sems_b, send_f, recv_f, send_r, recv_r,
             credit_f, credit_r):
        my = lax.axis_index("i")
        left = lax.rem(my + (N_DEV - 1), N_DEV)
        right = lax.rem(my + 1, N_DEV)

        barrier_sem = pltpu.get_barrier_semaphore()
        for nbr in (left, right):
            pl.semaphore_signal(
                barrier_sem, inc=1,
                device_id=(nbr,), device_id_type=pl.DeviceIdType.MESH,
            )
        pl.semaphore_wait(barrier_sem, 2)

        def a_dma(j):
            return pltpu.make_async_copy(
                a_hbm.at[pl.ds(j * a_rows, a_rows), :],
                sta_ref.at[j % 2], cp_sems_a.at[j % 2],
            )

        def b_dma(j):
            return pltpu.make_async_copy(
                b_hbm.at[pl.ds(j * b_rows, b_rows), :],
                stb_ref.at[j % 2], cp_sems_b.at[j % 2],
            )

        a_dma(0).start()
        a_dma(1).start()
        b_dma(0).start()
        b_dma(1).start()
        for j in range(N_CVT):
            sl = j % 2
            a_dma(j).wait()
            a16_ref[pl.ds(j * a_rows, a_rows), :] = (
                sta_ref[sl].astype(jnp.bfloat16))
            if j + 2 < N_CVT:
                a_dma(j + 2).start()
        for j in range(N_CVT):
            sl = j % 2
            b_dma(j).wait()
            b16_ref[pl.ds(j * b_rows, b_rows), :] = (
                stb_ref[sl].astype(jnp.bfloat16))
            if j + 2 < N_CVT:
                b_dma(j + 2).start()

        def rows(c):
            return pl.ds(c * M_CHUNK, M_CHUNK)

        def partial_f(c, dst):
            dst[...] = jnp.dot(
                a16_ref[rows(c), :], b16_ref[:, :nh],
                preferred_element_type=jnp.float32,
            )

        def partial_r(c, dst):
            dst[...] = jnp.dot(
                a16_ref[rows(c), :], b16_ref[:, nh:],
                preferred_element_type=jnp.float32,
            )

        partial_f(lax.rem(my + (N_DEV - 1), N_DEV), ppf_ref)
        partial_r(lax.rem(my + 1, N_DEV), ppr_ref)
        accf_ref[0] = ppf_ref[...].astype(jnp.bfloat16)
        accr_ref[0] = ppr_ref[...].astype(jnp.bfloat16)

        for s in range(N_DEV - 1):
            snd = s % 2
            rcv = (s + 1) % 2
            if s >= 2:
                pl.semaphore_wait(credit_f, 1)
                pl.semaphore_wait(credit_r, 1)
            rdma_f = pltpu.make_async_remote_copy(
                src_ref=accf_ref.at[snd], dst_ref=accf_ref.at[rcv],
                send_sem=send_f.at[snd], recv_sem=recv_f.at[rcv],
                device_id=(right,), device_id_type=pl.DeviceIdType.MESH,
            )
            rdma_r = pltpu.make_async_remote_copy(
                src_ref=accr_ref.at[snd], dst_ref=accr_ref.at[rcv],
                send_sem=send_r.at[snd], recv_sem=recv_r.at[rcv],
                device_id=(left,), device_id_type=pl.DeviceIdType.MESH,
            )
            rdma_f.start()
            rdma_r.start()
            cf = lax.rem(my + (2 * N_DEV - 2 - s), N_DEV)
            cr = lax.rem(my + 2 + s, N_DEV)
            partial_f(cf, ppf_ref)
            partial_r(cr, ppr_ref)
            rdma_f.wait()
            rdma_r.wait()
            pl.semaphore_signal(
                credit_f, inc=1,
                device_id=(left,), device_id_type=pl.DeviceIdType.MESH,
            )
            pl.semaphore_signal(
                credit_r, inc=1,
                device_id=(right,), device_id_type=pl.DeviceIdType.MESH,
            )
            if s < N_DEV - 2:
                accf_ref[rcv] = (
                    accf_ref[rcv].astype(jnp.float32) + ppf_ref[...]
                ).astype(jnp.bfloat16)
                accr_ref[rcv] = (
                    accr_ref[rcv].astype(jnp.float32) + ppr_ref[...]
                ).astype(jnp.bfloat16)
            else:
                out_ref[:, :nh] = accf_ref[rcv].astype(jnp.float32) + ppf_ref[...]
                out_ref[:, nh:] = accr_ref[rcv].astype(jnp.float32) + ppr_ref[...]

        pl.semaphore_wait(credit_f, 2)
        pl.semaphore_wait(credit_r, 2)

    return pl.pallas_call(
        body,
        out_shape=jax.ShapeDtypeStruct((M_CHUNK, n), jnp.float32),
        in_specs=[
            pl.BlockSpec(memory_space=pl.ANY),
            pl.BlockSpec(memory_space=pl.ANY),
        ],
        out_specs=pl.BlockSpec(memory_space=pltpu.VMEM),
        scratch_shapes=[
            pltpu.VMEM((m, k), jnp.bfloat16),
            pltpu.VMEM((k, n), jnp.bfloat16),
            pltpu.VMEM((2, m // N_CVT, k), jnp.float32),
            pltpu.VMEM((2, k // N_CVT, n), jnp.float32),
            pltpu.VMEM((2, M_CHUNK, nh), jnp.bfloat16),
            pltpu.VMEM((2, M_CHUNK, nh), jnp.bfloat16),
            pltpu.VMEM((M_CHUNK, nh), jnp.float32),
            pltpu.VMEM((M_CHUNK, nh), jnp.float32),
            pltpu.SemaphoreType.DMA((2,)),
            pltpu.SemaphoreType.DMA((2,)),
            pltpu.SemaphoreType.DMA((2,)),
            pltpu.SemaphoreType.DMA((2,)),
            pltpu.SemaphoreType.DMA((2,)),
            pltpu.SemaphoreType.DMA((2,)),
            pltpu.SemaphoreType.REGULAR,
            pltpu.SemaphoreType.REGULAR,
        ],
        compiler_params=pltpu.CompilerParams(
            collective_id=0,
            vmem_limit_bytes=100 * 1024 * 1024,
        ),
    )(A, B)


# device time: 133163 ns/iter; 2.9782x vs baseline; 1.0387x over previous
import jax
import jax.numpy as jnp
from jax import lax
from jax.experimental import pallas as pl
from jax.experimental.pallas import tpu as pltpu

N_DEV = 8
M = 3072
M_CHUNK = M // N_DEV

N_CVT_B = 8


def kernel(A, B):
    m, k = A.shape
    k2, n = B.shape
    assert m == M and k == k2
    nh = n // 2
    b_rows = k // N_CVT_B

    def body(a_hbm, b_hbm, out_ref, b16_ref, stb_ref, staf_ref, star_ref,
             af16_ref, ar16_ref, accf_ref, accr_ref, ppf_ref, ppr_ref,
             cp_sems_b, cp_sems_af, cp_sems_ar,
             send_f, recv_f, send_r, recv_r, credit_f, credit_r):
        my = lax.axis_index("i")
        left = lax.rem(my + (N_DEV - 1), N_DEV)
        right = lax.rem(my + 1, N_DEV)

        barrier_sem = pltpu.get_barrier_semaphore()
        for nbr in (left, right):
            pl.semaphore_signal(
                barrier_sem, inc=1,
                device_id=(nbr,), device_id_type=pl.DeviceIdType.MESH,
            )
        pl.semaphore_wait(barrier_sem, 2)

        def b_dma(j):
            return pltpu.make_async_copy(
                b_hbm.at[pl.ds(j * b_rows, b_rows), :],
                stb_ref.at[j % 2], cp_sems_b.at[j % 2],
            )

        def cF(t):
            return lax.rem(my + (2 * N_DEV - 1 - t), N_DEV)

        def cR(t):
            return lax.rem(my + 1 + t, N_DEV)

        def af_dma(t):
            return pltpu.make_async_copy(
                a_hbm.at[pl.ds(cF(t) * M_CHUNK, M_CHUNK), :],
                staf_ref.at[t % 2], cp_sems_af.at[t % 2],
            )

        def ar_dma(t):
            return pltpu.make_async_copy(
                a_hbm.at[pl.ds(cR(t) * M_CHUNK, M_CHUNK), :],
                star_ref.at[t % 2], cp_sems_ar.at[t % 2],
            )

        def a_step(t):
            sl = t % 2
            af_dma(t).wait()
            af16_ref[sl] = staf_ref[sl].astype(jnp.bfloat16)
            ar_dma(t).wait()
            ar16_ref[sl] = star_ref[sl].astype(jnp.bfloat16)
            if t + 2 < N_DEV:
                af_dma(t + 2).start()
                ar_dma(t + 2).start()

        b_dma(0).start()
        b_dma(1).start()
        af_dma(0).start()
        ar_dma(0).start()
        af_dma(1).start()
        ar_dma(1).start()
        for j in range(N_CVT_B):
            sl = j % 2
            b_dma(j).wait()
            b16_ref[pl.ds(j * b_rows, b_rows), :] = (
                stb_ref[sl].astype(jnp.bfloat16))
            if j + 2 < N_CVT_B:
                b_dma(j + 2).start()

        def partial_f(t, dst):
            dst[...] = jnp.dot(
                af16_ref[t % 2], b16_ref[:, :nh],
                preferred_element_type=jnp.float32,
            )

        def partial_r(t, dst):
            dst[...] = jnp.dot(
                ar16_ref[t % 2], b16_ref[:, nh:],
                preferred_element_type=jnp.float32,
            )

        a_step(0)
        partial_f(0, ppf_ref)
        partial_r(0, ppr_ref)
        accf_ref[0] = ppf_ref[...].astype(jnp.bfloat16)
        accr_ref[0] = ppr_ref[...].astype(jnp.bfloat16)

        for s in range(N_DEV - 1):
            t = s + 1
            snd = s % 2
            rcv = (s + 1) % 2
            if s >= 2:
                pl.semaphore_wait(credit_f, 1)
                pl.semaphore_wait(credit_r, 1)
            rdma_f = pltpu.make_async_remote_copy(
                src_ref=accf_ref.at[snd], dst_ref=accf_ref.at[rcv],
                send_sem=send_f.at[snd], recv_sem=recv_f.at[rcv],
                device_id=(right,), device_id_type=pl.DeviceIdType.MESH,
            )
            rdma_r = pltpu.make_async_remote_copy(
                src_ref=accr_ref.at[snd], dst_ref=accr_ref.at[rcv],
                send_sem=send_r.at[snd], recv_sem=recv_r.at[rcv],
                device_id=(left,), device_id_type=pl.DeviceIdType.MESH,
            )
            rdma_f.start()
            rdma_r.start()
            a_step(t)
            partial_f(t, ppf_ref)
            partial_r(t, ppr_ref)
            rdma_f.wait()
            rdma_r.wait()
            pl.semaphore_signal(
                credit_f, inc=1,
                device_id=(left,), device_id_type=pl.DeviceIdType.MESH,
            )
            pl.semaphore_signal(
                credit_r, inc=1,
                device_id=(right,), device_id_type=pl.DeviceIdType.MESH,
            )
            if s < N_DEV - 2:
                accf_ref[rcv] = (
                    accf_ref[rcv].astype(jnp.float32) + ppf_ref[...]
                ).astype(jnp.bfloat16)
                accr_ref[rcv] = (
                    accr_ref[rcv].astype(jnp.float32) + ppr_ref[...]
                ).astype(jnp.bfloat16)
            else:
                out_ref[:, :nh] = accf_ref[rcv].astype(jnp.float32) + ppf_ref[...]
                out_ref[:, nh:] = accr_ref[rcv].astype(jnp.float32) + ppr_ref[...]

        pl.semaphore_wait(credit_f, 2)
        pl.semaphore_wait(credit_r, 2)

    return pl.pallas_call(
        body,
        out_shape=jax.ShapeDtypeStruct((M_CHUNK, n), jnp.float32),
        in_specs=[
            pl.BlockSpec(memory_space=pl.ANY),
            pl.BlockSpec(memory_space=pl.ANY),
        ],
        out_specs=pl.BlockSpec(memory_space=pltpu.VMEM),
        scratch_shapes=[
            pltpu.VMEM((k, n), jnp.bfloat16),
            pltpu.VMEM((2, k // N_CVT_B, n), jnp.float32),
            pltpu.VMEM((2, M_CHUNK, k), jnp.float32),
            pltpu.VMEM((2, M_CHUNK, k), jnp.float32),
            pltpu.VMEM((2, M_CHUNK, k), jnp.bfloat16),
            pltpu.VMEM((2, M_CHUNK, k), jnp.bfloat16),
            pltpu.VMEM((2, M_CHUNK, nh), jnp.bfloat16),
            pltpu.VMEM((2, M_CHUNK, nh), jnp.bfloat16),
            pltpu.VMEM((M_CHUNK, nh), jnp.float32),
            pltpu.VMEM((M_CHUNK, nh), jnp.float32),
            pltpu.SemaphoreType.DMA((2,)),
            pltpu.SemaphoreType.DMA((2,)),
            pltpu.SemaphoreType.DMA((2,)),
            pltpu.SemaphoreType.DMA((2,)),
            pltpu.SemaphoreType.DMA((2,)),
            pltpu.SemaphoreType.DMA((2,)),
            pltpu.SemaphoreType.DMA((2,)),
            pltpu.SemaphoreType.REGULAR,
            pltpu.SemaphoreType.REGULAR,
        ],
        compiler_params=pltpu.CompilerParams(
            collective_id=0,
            vmem_limit_bytes=100 * 1024 * 1024,
        ),
    )(A, B)


# device time: 126722 ns/iter; 3.1295x vs baseline; 1.0508x over previous
import jax
import jax.numpy as jnp
from jax import lax
from jax.experimental import pallas as pl
from jax.experimental.pallas import tpu as pltpu

N_DEV = 8
M = 3072
M_CHUNK = M // N_DEV
N_THIRDS = 3

A_STEPS = 32
B_STEPS = 32


def kernel(A, B):
    m, k = A.shape
    k2, n = B.shape
    assert m == M and k == k2
    nw = n // N_THIRDS
    a_rows = M_CHUNK // 4
    b_rows = k // B_STEPS

    def body(a_hbm, b_hbm, out_hbm, a16_ref, b16_ref, sta_ref, stb_ref,
             sb1_ref, rb1_ref, w_ref, rb2_ref, rb3_ref, ob_ref,
             cp_sems_a, cp_sems_b, out_sems,
             s1, r1, s2, r2, s3, r3, exit_sem):
        my = lax.axis_index("i")

        def fx(p):
            return p + 1 - 2 * lax.rem(p, 2)

        def fy(p):
            p4 = lax.rem(p, 4)
            return p - p4 + 3 - p4

        def fz(p):
            return lax.rem(p + 4, N_DEV)

        def F(p, mask):
            bx = mask // 4
            by = lax.rem(mask, 4) // 2
            bz = lax.rem(mask, 2)
            p = bz * fz(p) + (1 - bz) * p
            p = by * fy(p) + (1 - by) * p
            p = bx * fx(p) + (1 - bx) * p
            return p

        axes = [(fx, fy, fz), (fy, fz, fx), (fz, fx, fy)]

        def w_chunk(g, mi):
            _, F1, F2 = axes[g]
            c = my
            if mi & 1:
                c = F2(c)
            if mi & 2:
                c = F1(c)
            return c

        def out_chunk(g, mi):
            return axes[g][0](w_chunk(g, mi))

        barrier_sem = pltpu.get_barrier_semaphore()
        for g in range(N_THIRDS):
            pl.semaphore_signal(
                barrier_sem, inc=1,
                device_id=(axes[g][0](my),),
                device_id_type=pl.DeviceIdType.MESH,
            )
        pl.semaphore_wait(barrier_sem, 3)

        def b_dma(j):
            return pltpu.make_async_copy(
                b_hbm.at[pl.ds(j * b_rows, b_rows), :],
                stb_ref.at[lax.rem(j, 2)], cp_sems_b.at[lax.rem(j, 2)],
            )

        b_dma(0).start()
        b_dma(1).start()

        def b_body(j, _):
            sl = lax.rem(j, 2)
            b_dma(j).wait()
            b16_ref[pl.ds(j * b_rows, b_rows), :] = (
                stb_ref[sl].astype(jnp.bfloat16))

            @pl.when(j + 2 < B_STEPS)
            def _():
                b_dma(j + 2).start()
            return 0

        lax.fori_loop(0, B_STEPS, b_body, 0)

        def a_chunk_of(t):
            return F(my, 7 - t // 4)

        def a_row(t):
            return a_chunk_of(t) * M_CHUNK + lax.rem(t, 4) * a_rows

        def a_dma(t):
            return pltpu.make_async_copy(
                a_hbm.at[pl.ds(a_row(t), a_rows), :],
                sta_ref.at[lax.rem(t, 2)], cp_sems_a.at[lax.rem(t, 2)],
            )

        def rows(c):
            return pl.ds(c * M_CHUNK, M_CHUNK)

        def pp16(c, g):
            return jnp.dot(
                a16_ref[rows(c), :], b16_ref[:, g * nw:(g + 1) * nw],
                preferred_element_type=jnp.float32,
            ).astype(jnp.bfloat16)

        def rdma1(g):
            return pltpu.make_async_remote_copy(
                src_ref=sb1_ref.at[g], dst_ref=rb1_ref.at[g],
                send_sem=s1.at[g], recv_sem=r1.at[g],
                device_id=(axes[g][0](my),),
                device_id_type=pl.DeviceIdType.MESH,
            )

        def send_phase1(g):
            for mi in range(4):
                sb1_ref[g, mi] = pp16(out_chunk(g, mi), g)
            rdma1(g).start()

        a_dma(0).start()
        a_dma(1).start()

        def a_body(t, _):
            sl = lax.rem(t, 2)
            a_dma(t).wait()
            a16_ref[pl.ds(a_row(t), a_rows), :] = (
                sta_ref[sl].astype(jnp.bfloat16))

            @pl.when(t + 2 < A_STEPS)
            def _():
                a_dma(t + 2).start()

            @pl.when(t == 15)
            def _():
                send_phase1(0)

            @pl.when(t == 23)
            def _():
                send_phase1(1)

            @pl.when(t == 27)
            def _():
                send_phase1(2)
            return 0

        lax.fori_loop(0, A_STEPS, a_body, 0)

        for g in range(N_THIRDS):
            for mi in range(4):
                w_ref[g, mi] = pp16(w_chunk(g, mi), g)

        for g in range(N_THIRDS):
            rdma1(g).wait()
            for mi in range(4):
                w_ref[g, mi] = (
                    w_ref[g, mi].astype(jnp.float32)
                    + rb1_ref[g, mi].astype(jnp.float32)
                ).astype(jnp.bfloat16)
            pltpu.make_async_remote_copy(
                src_ref=w_ref.at[g, pl.ds(2, 2)], dst_ref=rb2_ref.at[g],
                send_sem=s2.at[g], recv_sem=r2.at[g],
                device_id=(axes[g][1](my),),
                device_id_type=pl.DeviceIdType.MESH,
            ).start()

        for g in range(N_THIRDS):
            pltpu.make_async_remote_copy(
                src_ref=w_ref.at[g, pl.ds(2, 2)], dst_ref=rb2_ref.at[g],
                send_sem=s2.at[g], recv_sem=r2.at[g],
                device_id=(axes[g][1](my),),
                device_id_type=pl.DeviceIdType.MESH,
            ).wait()
            for j in range(2):
                w_ref[g, j] = (
                    w_ref[g, j].astype(jnp.float32)
                    + rb2_ref[g, j].astype(jnp.float32)
                ).astype(jnp.bfloat16)
            pltpu.make_async_remote_copy(
                src_ref=w_ref.at[g, pl.ds(1, 1)], dst_ref=rb3_ref.at[g],
                send_sem=s3.at[g], recv_sem=r3.at[g],
                device_id=(axes[g][2](my),),
                device_id_type=pl.DeviceIdType.MESH,
            ).start()

        for g in range(N_THIRDS):
            pltpu.make_async_remote_copy(
                src_ref=w_ref.at[g, pl.ds(1, 1)], dst_ref=rb3_ref.at[g],
                send_sem=s3.at[g], recv_sem=r3.at[g],
                device_id=(axes[g][2](my),),
                device_id_type=pl.DeviceIdType.MESH,
            ).wait()
            osl = g % 2
            if g >= 2:
                pltpu.make_async_copy(
                    ob_ref.at[osl],
                    out_hbm.at[:, pl.ds((g - 2) * nw, nw)],
                    out_sems.at[osl],
                ).wait()
            ob_ref[osl] = (
                w_ref[g, 0].astype(jnp.float32)
                + rb3_ref[g, 0].astype(jnp.float32)
            )
            pltpu.make_async_copy(
                ob_ref.at[osl], out_hbm.at[:, pl.ds(g * nw, nw)],
                out_sems.at[osl],
            ).start()
        for g in (1, 2):
            pltpu.make_async_copy(
                ob_ref.at[g % 2], out_hbm.at[:, pl.ds(g * nw, nw)],
                out_sems.at[g % 2],
            ).wait()

        for g in range(N_THIRDS):
            pl.semaphore_signal(
                exit_sem, inc=1,
                device_id=(axes[g][0](my),),
                device_id_type=pl.DeviceIdType.MESH,
            )
        pl.semaphore_wait(exit_sem, 3)

    return pl.pallas_call(
        body,
        out_shape=jax.ShapeDtypeStruct((M_CHUNK, n), jnp.float32),
        in_specs=[
            pl.BlockSpec(memory_space=pl.ANY),
            pl.BlockSpec(memory_space=pl.ANY),
        ],
        out_specs=pl.BlockSpec(memory_space=pl.ANY),
        scratch_shapes=[
            pltpu.VMEM((m, k), jnp.bfloat16),
            pltpu.VMEM((k, n), jnp.bfloat16),
            pltpu.VMEM((2, M_CHUNK // 4, k), jnp.float32),
            pltpu.VMEM((2, k // B_STEPS, n), jnp.float32),
            pltpu.VMEM((N_THIRDS, 4, M_CHUNK, nw), jnp.bfloat16),
            pltpu.VMEM((N_THIRDS, 4, M_CHUNK, nw), jnp.bfloat16),
            pltpu.VMEM((N_THIRDS, 4, M_CHUNK, nw), jnp.bfloat16),
            pltpu.VMEM((N_THIRDS, 2, M_CHUNK, nw), jnp.bfloat16),
            pltpu.VMEM((N_THIRDS, 1, M_CHUNK, nw), jnp.bfloat16),
            pltpu.VMEM((2, M_CHUNK, nw), jnp.float32),
            pltpu.SemaphoreType.DMA((2,)),
            pltpu.SemaphoreType.DMA((2,)),
            pltpu.SemaphoreType.DMA((2,)),
            pltpu.SemaphoreType.DMA((N_THIRDS,)),
            pltpu.SemaphoreType.DMA((N_THIRDS,)),
            pltpu.SemaphoreType.DMA((N_THIRDS,)),
            pltpu.SemaphoreType.DMA((N_THIRDS,)),
            pltpu.SemaphoreType.DMA((N_THIRDS,)),
            pltpu.SemaphoreType.DMA((N_THIRDS,)),
            pltpu.SemaphoreType.REGULAR,
        ],
        compiler_params=pltpu.CompilerParams(
            collective_id=0,
            vmem_limit_bytes=100 * 1024 * 1024,
        ),
    )(A, B)


# device time: 109434 ns/iter; 3.6239x vs baseline; 1.1580x over previous
import jax
import jax.numpy as jnp
from jax import lax
from jax.experimental import pallas as pl
from jax.experimental.pallas import tpu as pltpu

N_DEV = 8
M = 3072
M_CHUNK = M // N_DEV
N_THIRDS = 3

A_STEPS = 16
B_STEPS = 16


def kernel(A, B):
    m, k = A.shape
    k2, n = B.shape
    assert m == M and k == k2
    nw = n // N_THIRDS
    a_rows = M_CHUNK // 2
    b_rows = k // B_STEPS

    def body(a_hbm, b_hbm, out_hbm, a16_ref, b16_ref, sta_ref, stb_ref,
             sb1_ref, rb1_ref, w_ref, rb2_ref, rb3_ref, ob_ref,
             cp_sems_a, cp_sems_b, out_sems,
             s1, r1, s2, r2, s3, r3, exit_sem):
        my = lax.axis_index("i")

        def fx(p):
            return p + 1 - 2 * lax.rem(p, 2)

        def fy(p):
            p4 = lax.rem(p, 4)
            return p - p4 + 3 - p4

        def fz(p):
            return lax.rem(p + 4, N_DEV)

        def F(p, mask):
            bx = mask // 4
            by = lax.rem(mask, 4) // 2
            bz = lax.rem(mask, 2)
            p = bz * fz(p) + (1 - bz) * p
            p = by * fy(p) + (1 - by) * p
            p = bx * fx(p) + (1 - bx) * p
            return p

        axes = [(fx, fy, fz), (fy, fz, fx), (fz, fx, fy)]

        def w_chunk(g, mi):
            _, F1, F2 = axes[g]
            c = my
            if mi & 1:
                c = F2(c)
            if mi & 2:
                c = F1(c)
            return c

        def out_chunk(g, mi):
            return axes[g][0](w_chunk(g, mi))

        barrier_sem = pltpu.get_barrier_semaphore()
        for g in range(N_THIRDS):
            pl.semaphore_signal(
                barrier_sem, inc=1,
                device_id=(axes[g][0](my),),
                device_id_type=pl.DeviceIdType.MESH,
            )
        pl.semaphore_wait(barrier_sem, 3)

        def b_dma(j):
            return pltpu.make_async_copy(
                b_hbm.at[pl.ds(j * b_rows, b_rows), :],
                stb_ref.at[lax.rem(j, 2)], cp_sems_b.at[lax.rem(j, 2)],
            )

        def a_chunk_of(t):
            return F(my, 7 - t // 2)

        def a_row(t):
            return a_chunk_of(t) * M_CHUNK + lax.rem(t, 2) * a_rows

        def a_dma(t):
            return pltpu.make_async_copy(
                a_hbm.at[pl.ds(a_row(t), a_rows), :],
                sta_ref.at[lax.rem(t, 2)], cp_sems_a.at[lax.rem(t, 2)],
            )

        b_dma(0).start()
        b_dma(1).start()
        a_dma(0).start()
        a_dma(1).start()

        def b_body(j, _):
            sl = lax.rem(j, 2)
            b_dma(j).wait()
            b16_ref[pl.ds(j * b_rows, b_rows), :] = (
                stb_ref[sl].astype(jnp.bfloat16))

            @pl.when(j + 2 < B_STEPS)
            def _():
                b_dma(j + 2).start()
            return 0

        lax.fori_loop(0, B_STEPS, b_body, 0)

        def rows(c):
            return pl.ds(c * M_CHUNK, M_CHUNK)

        def pp16(c, g):
            return jnp.dot(
                a16_ref[rows(c), :], b16_ref[:, g * nw:(g + 1) * nw],
                preferred_element_type=jnp.float32,
            ).astype(jnp.bfloat16)

        def rdma1(g):
            return pltpu.make_async_remote_copy(
                src_ref=sb1_ref.at[g], dst_ref=rb1_ref.at[g],
                send_sem=s1.at[g], recv_sem=r1.at[g],
                device_id=(axes[g][0](my),),
                device_id_type=pl.DeviceIdType.MESH,
            )

        def send_phase1(g):
            for mi in range(4):
                sb1_ref[g, mi] = pp16(out_chunk(g, mi), g)
            rdma1(g).start()

        def a_body(t, _):
            sl = lax.rem(t, 2)
            a_dma(t).wait()
            a16_ref[pl.ds(a_row(t), a_rows), :] = (
                sta_ref[sl].astype(jnp.bfloat16))

            @pl.when(t + 2 < A_STEPS)
            def _():
                a_dma(t + 2).start()

            @pl.when(t == 7)
            def _():
                send_phase1(0)

            @pl.when(t == 11)
            def _():
                send_phase1(1)

            @pl.when(t == 13)
            def _():
                send_phase1(2)
            return 0

        lax.fori_loop(0, A_STEPS, a_body, 0)

        for g in range(N_THIRDS):
            for mi in range(4):
                w_ref[g, mi] = pp16(w_chunk(g, mi), g)

        def combine1(g, mi):
            w_ref[g, mi] = (
                w_ref[g, mi].astype(jnp.float32)
                + rb1_ref[g, mi].astype(jnp.float32)
            ).astype(jnp.bfloat16)

        for g in range(N_THIRDS):
            rdma1(g).wait()
            combine1(g, 2)
            combine1(g, 3)
            pltpu.make_async_remote_copy(
                src_ref=w_ref.at[g, pl.ds(2, 2)], dst_ref=rb2_ref.at[g],
                send_sem=s2.at[g], recv_sem=r2.at[g],
                device_id=(axes[g][1](my),),
                device_id_type=pl.DeviceIdType.MESH,
            ).start()
            combine1(g, 0)
            combine1(g, 1)

        def combine2(g, j):
            w_ref[g, j] = (
                w_ref[g, j].astype(jnp.float32)
                + rb2_ref[g, j].astype(jnp.float32)
            ).astype(jnp.bfloat16)

        for g in range(N_THIRDS):
            pltpu.make_async_remote_copy(
                src_ref=w_ref.at[g, pl.ds(2, 2)], dst_ref=rb2_ref.at[g],
                send_sem=s2.at[g], recv_sem=r2.at[g],
                device_id=(axes[g][1](my),),
                device_id_type=pl.DeviceIdType.MESH,
            ).wait()
            combine2(g, 1)
            pltpu.make_async_remote_copy(
                src_ref=w_ref.at[g, pl.ds(1, 1)], dst_ref=rb3_ref.at[g],
                send_sem=s3.at[g], recv_sem=r3.at[g],
                device_id=(axes[g][2](my),),
                device_id_type=pl.DeviceIdType.MESH,
            ).start()
            combine2(g, 0)

        for g in range(N_THIRDS):
            pltpu.make_async_remote_copy(
                src_ref=w_ref.at[g, pl.ds(1, 1)], dst_ref=rb3_ref.at[g],
                send_sem=s3.at[g], recv_sem=r3.at[g],
                device_id=(axes[g][2](my),),
                device_id_type=pl.DeviceIdType.MESH,
            ).wait()
            osl = g % 2
            if g >= 2:
                pltpu.make_async_copy(
                    ob_ref.at[osl],
                    out_hbm.at[:, pl.ds((g - 2) * nw, nw)],
                    out_sems.at[osl],
                ).wait()
            ob_ref[osl] = (
                w_ref[g, 0].astype(jnp.float32)
                + rb3_ref[g, 0].astype(jnp.float32)
            )
            pltpu.make_async_copy(
                ob_ref.at[osl], out_hbm.at[:, pl.ds(g * nw, nw)],
                out_sems.at[osl],
            ).start()
        for g in (1, 2):
            pltpu.make_async_copy(
                ob_ref.at[g % 2], out_hbm.at[:, pl.ds(g * nw, nw)],
                out_sems.at[g % 2],
            ).wait()

        for g in range(N_THIRDS):
            pl.semaphore_signal(
                exit_sem, inc=1,
                device_id=(axes[g][0](my),),
                device_id_type=pl.DeviceIdType.MESH,
            )
        pl.semaphore_wait(exit_sem, 3)

    return pl.pallas_call(
        body,
        out_shape=jax.ShapeDtypeStruct((M_CHUNK, n), jnp.float32),
        in_specs=[
            pl.BlockSpec(memory_space=pl.ANY),
            pl.BlockSpec(memory_space=pl.ANY),
        ],
        out_specs=pl.BlockSpec(memory_space=pl.ANY),
        scratch_shapes=[
            pltpu.VMEM((m, k), jnp.bfloat16),
            pltpu.VMEM((k, n), jnp.bfloat16),
            pltpu.VMEM((2, M_CHUNK // 2, k), jnp.float32),
            pltpu.VMEM((2, k // B_STEPS, n), jnp.float32),
            pltpu.VMEM((N_THIRDS, 4, M_CHUNK, nw), jnp.bfloat16),
            pltpu.VMEM((N_THIRDS, 4, M_CHUNK, nw), jnp.bfloat16),
            pltpu.VMEM((N_THIRDS, 4, M_CHUNK, nw), jnp.bfloat16),
            pltpu.VMEM((N_THIRDS, 2, M_CHUNK, nw), jnp.bfloat16),
            pltpu.VMEM((N_THIRDS, 1, M_CHUNK, nw), jnp.bfloat16),
            pltpu.VMEM((2, M_CHUNK, nw), jnp.float32),
            pltpu.SemaphoreType.DMA((2,)),
            pltpu.SemaphoreType.DMA((2,)),
            pltpu.SemaphoreType.DMA((2,)),
            pltpu.SemaphoreType.DMA((N_THIRDS,)),
            pltpu.SemaphoreType.DMA((N_THIRDS,)),
            pltpu.SemaphoreType.DMA((N_THIRDS,)),
            pltpu.SemaphoreType.DMA((N_THIRDS,)),
            pltpu.SemaphoreType.DMA((N_THIRDS,)),
            pltpu.SemaphoreType.DMA((N_THIRDS,)),
            pltpu.SemaphoreType.REGULAR,
        ],
        compiler_params=pltpu.CompilerParams(
            collective_id=0,
            vmem_limit_bytes=100 * 1024 * 1024,
        ),
    )(A, B)


# device time: 106423 ns/iter; 3.7265x vs baseline; 1.0283x over previous
import jax
import jax.numpy as jnp
from jax import lax
from jax.experimental import pallas as pl
from jax.experimental.pallas import tpu as pltpu

N_DEV = 8
M = 3072
M_CHUNK = M // N_DEV
N_THIRDS = 3

A_STEPS = 16
B_STEPS = 16


def kernel(A, B):
    m, k = A.shape
    k2, n = B.shape
    assert m == M and k == k2
    nw = n // N_THIRDS
    a_rows = M_CHUNK // 2
    b_rows = k // B_STEPS

    def body(a_hbm, b_hbm, out_hbm, a16_ref, b16_ref, sta_ref, stb_ref,
             sb1_ref, rb1_ref, w_ref, rb2_ref, rb3_ref, ob_ref,
             cp_sems_a, cp_sems_b, out_sems,
             s1, r1, s1b, r1b, s2, r2, s2b, r2b, s3, r3, exit_sem):
        my = lax.axis_index("i")

        def fx(p):
            return p + 1 - 2 * lax.rem(p, 2)

        def fy(p):
            p4 = lax.rem(p, 4)
            return p - p4 + 3 - p4

        def fz(p):
            return lax.rem(p + 4, N_DEV)

        def F(p, mask):
            bx = mask // 4
            by = lax.rem(mask, 4) // 2
            bz = lax.rem(mask, 2)
            p = bz * fz(p) + (1 - bz) * p
            p = by * fy(p) + (1 - by) * p
            p = bx * fx(p) + (1 - bx) * p
            return p

        axes = [(fx, fy, fz), (fy, fz, fx), (fz, fx, fy)]

        def w_chunk(g, mi):
            _, F1, F2 = axes[g]
            c = my
            if mi & 1:
                c = F2(c)
            if mi & 2:
                c = F1(c)
            return c

        def out_chunk(g, mi):
            return axes[g][0](w_chunk(g, mi))

        barrier_sem = pltpu.get_barrier_semaphore()
        for g in range(N_THIRDS):
            pl.semaphore_signal(
                barrier_sem, inc=1,
                device_id=(axes[g][0](my),),
                device_id_type=pl.DeviceIdType.MESH,
            )
        pl.semaphore_wait(barrier_sem, 3)

        def b_dma(j):
            return pltpu.make_async_copy(
                b_hbm.at[pl.ds(j * b_rows, b_rows), :],
                stb_ref.at[lax.rem(j, 2)], cp_sems_b.at[lax.rem(j, 2)],
            )

        def a_chunk_of(t):
            return F(my, 7 - t // 2)

        def a_row(t):
            return a_chunk_of(t) * M_CHUNK + lax.rem(t, 2) * a_rows

        def a_dma(t):
            return pltpu.make_async_copy(
                a_hbm.at[pl.ds(a_row(t), a_rows), :],
                sta_ref.at[lax.rem(t, 2)], cp_sems_a.at[lax.rem(t, 2)],
            )

        b_dma(0).start()
        b_dma(1).start()
        a_dma(0).start()
        a_dma(1).start()

        def b_body(j, _):
            sl = lax.rem(j, 2)
            b_dma(j).wait()
            b16_ref[pl.ds(j * b_rows, b_rows), :] = (
                stb_ref[sl].astype(jnp.bfloat16))

            @pl.when(j + 2 < B_STEPS)
            def _():
                b_dma(j + 2).start()
            return 0

        lax.fori_loop(0, B_STEPS, b_body, 0)

        def rows(c):
            return pl.ds(c * M_CHUNK, M_CHUNK)

        def pp16(c, g):
            return jnp.dot(
                a16_ref[rows(c), :], b16_ref[:, g * nw:(g + 1) * nw],
                preferred_element_type=jnp.float32,
            ).astype(jnp.bfloat16)

        def rdma1a(g):
            return pltpu.make_async_remote_copy(
                src_ref=sb1_ref.at[g, pl.ds(2, 2)],
                dst_ref=rb1_ref.at[g, pl.ds(2, 2)],
                send_sem=s1.at[g], recv_sem=r1.at[g],
                device_id=(axes[g][0](my),),
                device_id_type=pl.DeviceIdType.MESH,
            )

        def rdma1b(g):
            return pltpu.make_async_remote_copy(
                src_ref=sb1_ref.at[g, pl.ds(0, 2)],
                dst_ref=rb1_ref.at[g, pl.ds(0, 2)],
                send_sem=s1b.at[g], recv_sem=r1b.at[g],
                device_id=(axes[g][0](my),),
                device_id_type=pl.DeviceIdType.MESH,
            )

        def send_phase1(g):
            sb1_ref[g, 2] = pp16(out_chunk(g, 2), g)
            sb1_ref[g, 3] = pp16(out_chunk(g, 3), g)
            rdma1a(g).start()
            sb1_ref[g, 0] = pp16(out_chunk(g, 0), g)
            sb1_ref[g, 1] = pp16(out_chunk(g, 1), g)
            rdma1b(g).start()

        def a_body(t, _):
            sl = lax.rem(t, 2)
            a_dma(t).wait()
            a16_ref[pl.ds(a_row(t), a_rows), :] = (
                sta_ref[sl].astype(jnp.bfloat16))

            @pl.when(t + 2 < A_STEPS)
            def _():
                a_dma(t + 2).start()

            @pl.when(t == 7)
            def _():
                send_phase1(0)

            @pl.when(t == 11)
            def _():
                send_phase1(1)

            @pl.when(t == 13)
            def _():
                send_phase1(2)
            return 0

        lax.fori_loop(0, A_STEPS, a_body, 0)

        for g in range(N_THIRDS):
            for mi in range(4):
                w_ref[g, mi] = pp16(w_chunk(g, mi), g)

        def combine1(g, mi):
            w_ref[g, mi] = (
                w_ref[g, mi].astype(jnp.float32)
                + rb1_ref[g, mi].astype(jnp.float32)
            ).astype(jnp.bfloat16)

        def rdma2a(g):
            return pltpu.make_async_remote_copy(
                src_ref=w_ref.at[g, pl.ds(3, 1)],
                dst_ref=rb2_ref.at[g, pl.ds(1, 1)],
                send_sem=s2.at[g], recv_sem=r2.at[g],
                device_id=(axes[g][1](my),),
                device_id_type=pl.DeviceIdType.MESH,
            )

        def rdma2b(g):
            return pltpu.make_async_remote_copy(
                src_ref=w_ref.at[g, pl.ds(2, 1)],
                dst_ref=rb2_ref.at[g, pl.ds(0, 1)],
                send_sem=s2b.at[g], recv_sem=r2b.at[g],
                device_id=(axes[g][1](my),),
                device_id_type=pl.DeviceIdType.MESH,
            )

        for g in range(N_THIRDS):
            rdma1a(g).wait()
            combine1(g, 3)
            rdma2a(g).start()
            combine1(g, 2)
            rdma2b(g).start()
            rdma1b(g).wait()
            combine1(g, 1)
            combine1(g, 0)

        def combine2(g, j):
            w_ref[g, j] = (
                w_ref[g, j].astype(jnp.float32)
                + rb2_ref[g, j].astype(jnp.float32)
            ).astype(jnp.bfloat16)

        for g in range(N_THIRDS):
            rdma2a(g).wait()
            combine2(g, 1)
            pltpu.make_async_remote_copy(
                src_ref=w_ref.at[g, pl.ds(1, 1)], dst_ref=rb3_ref.at[g],
                send_sem=s3.at[g], recv_sem=r3.at[g],
                device_id=(axes[g][2](my),),
                device_id_type=pl.DeviceIdType.MESH,
            ).start()
            rdma2b(g).wait()
            combine2(g, 0)

        for g in range(N_THIRDS):
            pltpu.make_async_remote_copy(
                src_ref=w_ref.at[g, pl.ds(1, 1)], dst_ref=rb3_ref.at[g],
                send_sem=s3.at[g], recv_sem=r3.at[g],
                device_id=(axes[g][2](my),),
                device_id_type=pl.DeviceIdType.MESH,
            ).wait()
            osl = g % 2
            if g >= 2:
                pltpu.make_async_copy(
                    ob_ref.at[osl],
                    out_hbm.at[:, pl.ds((g - 2) * nw, nw)],
                    out_sems.at[osl],
                ).wait()
            ob_ref[osl] = (
                w_ref[g, 0].astype(jnp.float32)
                + rb3_ref[g, 0].astype(jnp.float32)
            )
            pltpu.make_async_copy(
                ob_ref.at[osl], out_hbm.at[:, pl.ds(g * nw, nw)],
                out_sems.at[osl],
            ).start()
        for g in (1, 2):
            pltpu.make_async_copy(
                ob_ref.at[g % 2], out_hbm.at[:, pl.ds(g * nw, nw)],
                out_sems.at[g % 2],
            ).wait()

        for g in range(N_THIRDS):
            pl.semaphore_signal(
                exit_sem, inc=1,
                device_id=(axes[g][0](my),),
                device_id_type=pl.DeviceIdType.MESH,
            )
        pl.semaphore_wait(exit_sem, 3)

    return pl.pallas_call(
        body,
        out_shape=jax.ShapeDtypeStruct((M_CHUNK, n), jnp.float32),
        in_specs=[
            pl.BlockSpec(memory_space=pl.ANY),
            pl.BlockSpec(memory_space=pl.ANY),
        ],
        out_specs=pl.BlockSpec(memory_space=pl.ANY),
        scratch_shapes=[
            pltpu.VMEM((m, k), jnp.bfloat16),
            pltpu.VMEM((k, n), jnp.bfloat16),
            pltpu.VMEM((2, M_CHUNK // 2, k), jnp.float32),
            pltpu.VMEM((2, k // B_STEPS, n), jnp.float32),
            pltpu.VMEM((N_THIRDS, 4, M_CHUNK, nw), jnp.bfloat16),
            pltpu.VMEM((N_THIRDS, 4, M_CHUNK, nw), jnp.bfloat16),
            pltpu.VMEM((N_THIRDS, 4, M_CHUNK, nw), jnp.bfloat16),
            pltpu.VMEM((N_THIRDS, 2, M_CHUNK, nw), jnp.bfloat16),
            pltpu.VMEM((N_THIRDS, 1, M_CHUNK, nw), jnp.bfloat16),
            pltpu.VMEM((2, M_CHUNK, nw), jnp.float32),
            pltpu.SemaphoreType.DMA((2,)),
            pltpu.SemaphoreType.DMA((2,)),
            pltpu.SemaphoreType.DMA((2,)),
            pltpu.SemaphoreType.DMA((N_THIRDS,)),
            pltpu.SemaphoreType.DMA((N_THIRDS,)),
            pltpu.SemaphoreType.DMA((N_THIRDS,)),
            pltpu.SemaphoreType.DMA((N_THIRDS,)),
            pltpu.SemaphoreType.DMA((N_THIRDS,)),
            pltpu.SemaphoreType.DMA((N_THIRDS,)),
            pltpu.SemaphoreType.DMA((N_THIRDS,)),
            pltpu.SemaphoreType.DMA((N_THIRDS,)),
            pltpu.SemaphoreType.DMA((N_THIRDS,)),
            pltpu.SemaphoreType.DMA((N_THIRDS,)),
            pltpu.SemaphoreType.REGULAR,
        ],
        compiler_params=pltpu.CompilerParams(
            collective_id=0,
            vmem_limit_bytes=100 * 1024 * 1024,
        ),
    )(A, B)


# device time: 99917 ns/iter; 3.9691x vs baseline; 1.0651x over previous
import jax
import jax.numpy as jnp
from jax import lax
from jax.experimental import pallas as pl
from jax.experimental.pallas import tpu as pltpu

N_DEV = 8
M = 3072
M_CHUNK = M // N_DEV
N_THIRDS = 3

A_STEPS = 16
B_STEPS = 16


def kernel(A, B):
    m, k = A.shape
    k2, n = B.shape
    assert m == M and k == k2
    nw = n // N_THIRDS
    a_rows = M_CHUNK // 2
    b_rows = k // B_STEPS

    def body(a_hbm, b_hbm, out_hbm, a16_ref, b16_ref, sta_ref, stb_ref,
             sb1_ref, rb1_ref, w_ref, rb2_ref, rb3_ref, ob_ref,
             cp_sems_a, cp_sems_b, out_sems,
             s1, r1, s1b, r1b, s2, r2, s2b, r2b, s3, r3, exit_sem):
        my = lax.axis_index("i")

        def fx(p):
            return p + 1 - 2 * lax.rem(p, 2)

        def fy(p):
            p4 = lax.rem(p, 4)
            return p - p4 + 3 - p4

        def fz(p):
            return lax.rem(p + 4, N_DEV)

        def F(p, mask):
            bx = mask // 4
            by = lax.rem(mask, 4) // 2
            bz = lax.rem(mask, 2)
            p = bz * fz(p) + (1 - bz) * p
            p = by * fy(p) + (1 - by) * p
            p = bx * fx(p) + (1 - bx) * p
            return p

        axes = [(fx, fy, fz), (fy, fz, fx), (fz, fx, fy)]

        def w_chunk(g, mi):
            _, F1, F2 = axes[g]
            c = my
            if mi & 1:
                c = F2(c)
            if mi & 2:
                c = F1(c)
            return c

        def out_chunk(g, mi):
            return axes[g][0](w_chunk(g, mi))

        barrier_sem = pltpu.get_barrier_semaphore()
        for g in range(N_THIRDS):
            pl.semaphore_signal(
                barrier_sem, inc=1,
                device_id=(axes[g][0](my),),
                device_id_type=pl.DeviceIdType.MESH,
            )
        pl.semaphore_wait(barrier_sem, 3)

        def b_dma(j):
            return pltpu.make_async_copy(
                b_hbm.at[pl.ds(j * b_rows, b_rows), :],
                stb_ref.at[lax.rem(j, 2)], cp_sems_b.at[lax.rem(j, 2)],
            )

        def a_chunk_of(t):
            return F(my, 7 - t // 2)

        def a_row(t):
            return a_chunk_of(t) * M_CHUNK + lax.rem(t, 2) * a_rows

        def a_dma(t):
            return pltpu.make_async_copy(
                a_hbm.at[pl.ds(a_row(t), a_rows), :],
                sta_ref.at[lax.rem(t, 2)], cp_sems_a.at[lax.rem(t, 2)],
            )

        b_dma(0).start()
        b_dma(1).start()
        a_dma(0).start()
        a_dma(1).start()

        def b_body(j, _):
            sl = lax.rem(j, 2)
            b_dma(j).wait()
            b16_ref[pl.ds(j * b_rows, b_rows), :] = (
                stb_ref[sl].astype(jnp.bfloat16))

            @pl.when(j + 2 < B_STEPS)
            def _():
                b_dma(j + 2).start()
            return 0

        lax.fori_loop(0, B_STEPS, b_body, 0)

        def rows(c):
            return pl.ds(c * M_CHUNK, M_CHUNK)

        def pp16(c, g):
            return jnp.dot(
                a16_ref[rows(c), :], b16_ref[:, g * nw:(g + 1) * nw],
                preferred_element_type=jnp.float32,
            ).astype(jnp.bfloat16)

        def rdma1a(g):
            return pltpu.make_async_remote_copy(
                src_ref=sb1_ref.at[g, pl.ds(2, 2)],
                dst_ref=rb1_ref.at[g, pl.ds(2, 2)],
                send_sem=s1.at[g], recv_sem=r1.at[g],
                device_id=(axes[g][0](my),),
                device_id_type=pl.DeviceIdType.MESH,
            )

        def rdma1b(g):
            return pltpu.make_async_remote_copy(
                src_ref=sb1_ref.at[g, pl.ds(0, 2)],
                dst_ref=rb1_ref.at[g, pl.ds(0, 2)],
                send_sem=s1b.at[g], recv_sem=r1b.at[g],
                device_id=(axes[g][0](my),),
                device_id_type=pl.DeviceIdType.MESH,
            )

        def send1a(g):
            sb1_ref[g, 2] = pp16(out_chunk(g, 2), g)
            sb1_ref[g, 3] = pp16(out_chunk(g, 3), g)
            rdma1a(g).start()

        def send1b(g):
            sb1_ref[g, 0] = pp16(out_chunk(g, 0), g)
            sb1_ref[g, 1] = pp16(out_chunk(g, 1), g)
            rdma1b(g).start()

        fire = {3: ('a', 0), 5: ('a', 2), 7: ('b', 0),
                9: ('a', 1), 11: ('b', 1), 13: ('b', 2)}

        def a_body(t, _):
            sl = lax.rem(t, 2)
            a_dma(t).wait()
            a16_ref[pl.ds(a_row(t), a_rows), :] = (
                sta_ref[sl].astype(jnp.bfloat16))

            @pl.when(t + 2 < A_STEPS)
            def _():
                a_dma(t + 2).start()

            for trip, (half, g) in fire.items():
                @pl.when(t == trip)
                def _(half=half, g=g):
                    (send1a if half == 'a' else send1b)(g)
            return 0

        lax.fori_loop(0, A_STEPS, a_body, 0)

        def combine1(g, mi):
            w_ref[g, mi] = (
                w_ref[g, mi].astype(jnp.float32)
                + rb1_ref[g, mi].astype(jnp.float32)
            ).astype(jnp.bfloat16)

        def rdma2a(g):
            return pltpu.make_async_remote_copy(
                src_ref=w_ref.at[g, pl.ds(3, 1)],
                dst_ref=rb2_ref.at[g, pl.ds(1, 1)],
                send_sem=s2.at[g], recv_sem=r2.at[g],
                device_id=(axes[g][1](my),),
                device_id_type=pl.DeviceIdType.MESH,
            )

        def rdma2b(g):
            return pltpu.make_async_remote_copy(
                src_ref=w_ref.at[g, pl.ds(2, 1)],
                dst_ref=rb2_ref.at[g, pl.ds(0, 1)],
                send_sem=s2b.at[g], recv_sem=r2b.at[g],
                device_id=(axes[g][1](my),),
                device_id_type=pl.DeviceIdType.MESH,
            )

        for g in range(N_THIRDS):
            for mi in (3, 2, 1, 0):
                w_ref[g, mi] = pp16(w_chunk(g, mi), g)
            rdma1a(g).wait()
            combine1(g, 3)
            rdma2a(g).start()
            combine1(g, 2)
            rdma2b(g).start()

        for g in range(N_THIRDS):
            rdma1b(g).wait()
            combine1(g, 1)
            combine1(g, 0)

        def combine2(g, j):
            w_ref[g, j] = (
                w_ref[g, j].astype(jnp.float32)
                + rb2_ref[g, j].astype(jnp.float32)
            ).astype(jnp.bfloat16)

        for g in range(N_THIRDS):
            rdma2a(g).wait()
            combine2(g, 1)
            pltpu.make_async_remote_copy(
                src_ref=w_ref.at[g, pl.ds(1, 1)], dst_ref=rb3_ref.at[g],
                send_sem=s3.at[g], recv_sem=r3.at[g],
                device_id=(axes[g][2](my),),
                device_id_type=pl.DeviceIdType.MESH,
            ).start()
            rdma2b(g).wait()
            combine2(g, 0)

        for g in range(N_THIRDS):
            pltpu.make_async_remote_copy(
                src_ref=w_ref.at[g, pl.ds(1, 1)], dst_ref=rb3_ref.at[g],
                send_sem=s3.at[g], recv_sem=r3.at[g],
                device_id=(axes[g][2](my),),
                device_id_type=pl.DeviceIdType.MESH,
            ).wait()
            osl = g % 2
            if g >= 2:
                pltpu.make_async_copy(
                    ob_ref.at[osl],
                    out_hbm.at[:, pl.ds((g - 2) * nw, nw)],
                    out_sems.at[osl],
                ).wait()
            ob_ref[osl] = (
                w_ref[g, 0].astype(jnp.float32)
                + rb3_ref[g, 0].astype(jnp.float32)
            )
            pltpu.make_async_copy(
                ob_ref.at[osl], out_hbm.at[:, pl.ds(g * nw, nw)],
                out_sems.at[osl],
            ).start()
        for g in (1, 2):
            pltpu.make_async_copy(
                ob_ref.at[g % 2], out_hbm.at[:, pl.ds(g * nw, nw)],
                out_sems.at[g % 2],
            ).wait()

        for g in range(N_THIRDS):
            pl.semaphore_signal(
                exit_sem, inc=1,
                device_id=(axes[g][0](my),),
                device_id_type=pl.DeviceIdType.MESH,
            )
        pl.semaphore_wait(exit_sem, 3)

    return pl.pallas_call(
        body,
        out_shape=jax.ShapeDtypeStruct((M_CHUNK, n), jnp.float32),
        in_specs=[
            pl.BlockSpec(memory_space=pl.ANY),
            pl.BlockSpec(memory_space=pl.ANY),
        ],
        out_specs=pl.BlockSpec(memory_space=pl.ANY),
        scratch_shapes=[
            pltpu.VMEM((m, k), jnp.bfloat16),
            pltpu.VMEM((k, n), jnp.bfloat16),
            pltpu.VMEM((2, M_CHUNK // 2, k), jnp.float32),
            pltpu.VMEM((2, k // B_STEPS, n), jnp.float32),
            pltpu.VMEM((N_THIRDS, 4, M_CHUNK, nw), jnp.bfloat16),
            pltpu.VMEM((N_THIRDS, 4, M_CHUNK, nw), jnp.bfloat16),
            pltpu.VMEM((N_THIRDS, 4, M_CHUNK, nw), jnp.bfloat16),
            pltpu.VMEM((N_THIRDS, 2, M_CHUNK, nw), jnp.bfloat16),
            pltpu.VMEM((N_THIRDS, 1, M_CHUNK, nw), jnp.bfloat16),
            pltpu.VMEM((2, M_CHUNK, nw), jnp.float32),
            pltpu.SemaphoreType.DMA((2,)),
            pltpu.SemaphoreType.DMA((2,)),
            pltpu.SemaphoreType.DMA((2,)),
            pltpu.SemaphoreType.DMA((N_THIRDS,)),
            pltpu.SemaphoreType.DMA((N_THIRDS,)),
            pltpu.SemaphoreType.DMA((N_THIRDS,)),
            pltpu.SemaphoreType.DMA((N_THIRDS,)),
            pltpu.SemaphoreType.DMA((N_THIRDS,)),
            pltpu.SemaphoreType.DMA((N_THIRDS,)),
            pltpu.SemaphoreType.DMA((N_THIRDS,)),
            pltpu.SemaphoreType.DMA((N_THIRDS,)),
            pltpu.SemaphoreType.DMA((N_THIRDS,)),
            pltpu.SemaphoreType.DMA((N_THIRDS,)),
            pltpu.SemaphoreType.REGULAR,
        ],
        compiler_params=pltpu.CompilerParams(
            collective_id=0,
            vmem_limit_bytes=100 * 1024 * 1024,
        ),
    )(A, B)
